# baseline (speedup 1.0000x reference)
import functools

import jax
import jax.numpy as jnp
from jax.experimental import pallas as pl
from jax.experimental.pallas import tpu as pltpu

NBITS = 16
HIDDEN = 128
BATCH = 32
T = 8192

B_BLK = 8
T_BLK = 1024


def _bias_body(msg_ref, emb_ref, bias_ref):
    even = emb_ref[:, 0, :]
    odd = emb_ref[:, 1, :]
    base = jnp.sum(even, axis=0)
    diff = odd - even
    bias_ref[...] = (
        jnp.dot(msg_ref[...], diff, preferred_element_type=jnp.float32)
        + base[None, :]
    )


def _add_body(bias_ref, hid_ref, out_ref):
    out_ref[...] = hid_ref[...] + bias_ref[...][:, :, None]


@functools.partial(jax.jit, donate_argnums=())
def kernel(hidden, msg, emb_table):
    msg_f = msg.astype(jnp.float32)
    emb3 = emb_table.reshape(NBITS, 2, HIDDEN)

    bias = pl.pallas_call(
        _bias_body,
        out_shape=jax.ShapeDtypeStruct((BATCH, HIDDEN), jnp.float32),
    )(msg_f, emb3)

    grid = (BATCH // B_BLK, T // T_BLK)
    out = pl.pallas_call(
        _add_body,
        grid=grid,
        in_specs=[
            pl.BlockSpec((B_BLK, HIDDEN), lambda b, t: (b, 0)),
            pl.BlockSpec((B_BLK, HIDDEN, T_BLK), lambda b, t: (b, 0, t)),
        ],
        out_specs=pl.BlockSpec((B_BLK, HIDDEN, T_BLK), lambda b, t: (b, 0, t)),
        out_shape=jax.ShapeDtypeStruct((BATCH, HIDDEN, T), jnp.float32),
        compiler_params=pltpu.CompilerParams(
            dimension_semantics=("parallel", "parallel"),
        ),
    )(bias, hidden)
    return out


# speedup vs baseline: 1.0394x; 1.0394x over previous
import functools

import jax
import jax.numpy as jnp
from jax.experimental import pallas as pl
from jax.experimental.pallas import tpu as pltpu

NBITS = 16
HIDDEN = 128
BATCH = 32
T = 8192

B_BLK = 8
T_BLK = 2048


def _body(msg_ref, emb_ref, hid_ref, out_ref, bias_ref):
    b = pl.program_id(0)
    t = pl.program_id(1)

    @pl.when(jnp.logical_and(b == 0, t == 0))
    def _():
        even = emb_ref[:, 0, :]
        odd = emb_ref[:, 1, :]
        diff = odd - even
        base = jnp.sum(even, axis=0)
        bias_ref[...] = (
            jax.lax.dot(msg_ref[...], diff,
                        preferred_element_type=jnp.float32)
            + base[None, :]
        )

    blk_bias = bias_ref[pl.ds(b * B_BLK, B_BLK), :]
    out_ref[...] = hid_ref[...] + blk_bias[:, :, None]


@functools.partial(jax.jit, donate_argnums=())
def kernel(hidden, msg, emb_table):
    msg_f = msg.astype(jnp.float32)
    emb3 = emb_table.reshape(NBITS, 2, HIDDEN)

    grid = (BATCH // B_BLK, T // T_BLK)
    out = pl.pallas_call(
        _body,
        grid=grid,
        in_specs=[
            pl.BlockSpec((BATCH, NBITS), lambda b, t: (0, 0)),
            pl.BlockSpec((NBITS, 2, HIDDEN), lambda b, t: (0, 0, 0)),
            pl.BlockSpec((B_BLK, HIDDEN, T_BLK), lambda b, t: (b, 0, t)),
        ],
        out_specs=pl.BlockSpec((B_BLK, HIDDEN, T_BLK), lambda b, t: (b, 0, t)),
        out_shape=jax.ShapeDtypeStruct((BATCH, HIDDEN, T), jnp.float32),
        scratch_shapes=[pltpu.VMEM((BATCH, HIDDEN), jnp.float32)],
        compiler_params=pltpu.CompilerParams(
            dimension_semantics=("arbitrary", "arbitrary"),
        ),
    )(msg_f, emb3, hidden)
    return out
